# baseline (speedup 1.0000x reference)
import functools

import jax
import jax.numpy as jnp
from jax import lax
from jax.experimental import pallas as pl
from jax.experimental.pallas import tpu as pltpu
from jax.experimental.pallas import tpu_sc as plsc

_BATCH = 4096
_HIST = 50
_D = 128
_B = _BATCH * _HIST
_NW = 32
_BPW = _B // _NW
_C = 256
_NCHUNK = _BPW // _C
_NBUF = 3


def _sc_gather(idx_hbm, table_hbm, out_hbm, idx_v, rows_v, tab_s, sem_g, sem_s):
    sid = lax.axis_index("s")
    wid = sid * 2 + lax.axis_index("c")

    @pl.when(sid == 0)
    def _():
        pltpu.sync_copy(table_hbm, tab_s)
    pltpu.sync_copy(idx_hbm.at[wid], idx_v)
    plsc.subcore_barrier()
    base = wid * _BPW

    def g_copy(c, b):
        return pltpu.make_async_copy(
            tab_s.at[idx_v.at[pl.ds(c * _C, _C)]],
            rows_v.at[b],
            sem_g.at[b],
        )

    def s_copy(c, b):
        return pltpu.make_async_copy(
            rows_v.at[b],
            out_hbm.at[pl.ds(base + c * _C, _C)],
            sem_s.at[b],
        )

    g_copy(0, 0).start()
    g_copy(1, 1).start()

    def step(c, carry):
        b = lax.rem(c, _NBUF)

        @pl.when(c + 2 < _NCHUNK)
        def _():
            bn = lax.rem(c + 2, _NBUF)

            @pl.when(c >= _NBUF - 2)
            def _():
                s_copy(c + 2 - _NBUF, bn).wait()

            g_copy(c + 2, bn).start()

        g_copy(c, b).wait()
        s_copy(c, b).start()
        return carry

    lax.fori_loop(0, _NCHUNK, step, 0)
    for k in range(_NCHUNK - _NBUF, _NCHUNK):
        s_copy(k, k % _NBUF).wait()


@jax.jit
def _run(indices_t_flat, embed_word):
    mesh = plsc.VectorSubcoreMesh(core_axis_name="c", subcore_axis_name="s")
    fn = pl.kernel(
        _sc_gather,
        out_type=jax.ShapeDtypeStruct((_B, _D), jnp.float32),
        mesh=mesh,
        scratch_types=[
            pltpu.VMEM((_BPW,), jnp.int32),
            pltpu.VMEM((_NBUF, _C, _D), jnp.float32),
            pltpu.VMEM_SHARED((1002, _D), jnp.float32),
            pltpu.SemaphoreType.DMA((_NBUF,)),
            pltpu.SemaphoreType.DMA((_NBUF,)),
        ],
    )
    return fn(indices_t_flat, embed_word)


def kernel(indices, embed_word):
    idx_t = indices.T.reshape(_NW, _BPW)
    out_t = _run(idx_t, embed_word)
    out_t = out_t.reshape(_HIST, _BATCH, _D)
    return out_t.transpose(1, 0, 2)


# speedup vs baseline: 1.0081x; 1.0081x over previous
import jax
import jax.numpy as jnp
from jax import lax
from jax.experimental import pallas as pl
from jax.experimental.pallas import tpu as pltpu
from jax.experimental.pallas import tpu_sc as plsc

_BATCH = 4096
_HIST = 50
_D = 128
_B = _BATCH * _HIST
_NW = 32
_BPW = _B // _NW
_C = 128
_NCHUNK = _BPW // _C
_NBUF = 6


def _sc_gather(idx_hbm, table_hbm, out_hbm, idx_v, rows_v, tab_s, sem_g, sem_s):
    sid = lax.axis_index("s")
    wid = sid * 2 + lax.axis_index("c")

    @pl.when(sid == 0)
    def _():
        pltpu.sync_copy(table_hbm, tab_s)
    pltpu.sync_copy(idx_hbm.at[wid], idx_v)
    plsc.subcore_barrier()
    base = wid * _BPW

    def g_copy(c, b):
        return pltpu.make_async_copy(
            tab_s.at[idx_v.at[pl.ds(c * _C, _C)]],
            rows_v.at[b],
            sem_g.at[b],
        )

    def s_copy(c, b):
        return pltpu.make_async_copy(
            rows_v.at[b],
            out_hbm.at[pl.ds(base + c * _C, _C)],
            sem_s.at[b],
        )

    g_copy(0, 0).start()
    g_copy(1, 1).start()

    def step(c, carry):
        b = lax.rem(c, _NBUF)

        @pl.when(c + 2 < _NCHUNK)
        def _():
            bn = lax.rem(c + 2, _NBUF)

            @pl.when(c >= _NBUF - 2)
            def _():
                s_copy(c + 2 - _NBUF, bn).wait()

            g_copy(c + 2, bn).start()

        g_copy(c, b).wait()
        s_copy(c, b).start()
        return carry

    lax.fori_loop(0, _NCHUNK, step, 0)
    for k in range(_NCHUNK - _NBUF, _NCHUNK):
        s_copy(k, k % _NBUF).wait()


@jax.jit
def _run(indices_t_flat, embed_word):
    mesh = plsc.VectorSubcoreMesh(core_axis_name="c", subcore_axis_name="s")
    fn = pl.kernel(
        _sc_gather,
        out_type=jax.ShapeDtypeStruct((_B, _D), jnp.float32),
        mesh=mesh,
        scratch_types=[
            pltpu.VMEM((_BPW,), jnp.int32),
            pltpu.VMEM((_NBUF, _C, _D), jnp.float32),
            pltpu.VMEM_SHARED((1002, _D), jnp.float32),
            pltpu.SemaphoreType.DMA((_NBUF,)),
            pltpu.SemaphoreType.DMA((_NBUF,)),
        ],
    )
    return fn(indices_t_flat, embed_word)


def kernel(indices, embed_word):
    idx_t = indices.T.reshape(_NW, _BPW)
    out_t = _run(idx_t, embed_word)
    out_t = out_t.reshape(_HIST, _BATCH, _D)
    return out_t.transpose(1, 0, 2)
